# Initial kernel scaffold; baseline (speedup 1.0000x reference)
import jax
import jax.numpy as jnp
from jax.experimental import pallas as pl


def kernel(x, edge_index, edge_attr, batch, W1, b1, W2, b2):
    raise NotImplementedError("write your pallas kernel here")


# speedup vs baseline: 18.0719x; 18.0719x over previous
import functools

import jax
import jax.numpy as jnp
from jax import lax
from jax.experimental import pallas as pl
from jax.experimental.pallas import tpu as pltpu
from jax.experimental.pallas import tpu_sc as plsc

NUM_GRAPHS = 64
N_NODES = 10000
N_EDGES = 640000
D_FEAT = 128
D_EDGE = 16
HIDDEN = 256
OUT_DIM = 128

NW = 32
NODE_CHUNK = 80
NODE_NCHUNKS = N_NODES // NODE_CHUNK
EDGES_PER_W = N_EDGES // NW
EDGE_CHUNK = 2000
EDGE_NCHUNKS = EDGES_PER_W // EDGE_CHUNK


def _sc_pool_body(x_hbm, ei_hbm, ea_hbm, b_hbm,
                  nsum_hbm, ncnt_hbm, esum_hbm, ecnt_hbm,
                  btbl, xbuf, bbuf, srcbuf, eabuf,
                  nacc, ncntv, eacc, ecntv):
    cid = lax.axis_index("c")
    sid = lax.axis_index("s")
    wid = sid * 2 + cid

    zero16 = jnp.zeros((16,), jnp.float32)
    ones16 = jnp.ones((16,), jnp.float32)
    lane = lax.iota(jnp.int32, 16)

    def zrow(i, c):
        for j in range(D_FEAT // 16):
            nacc[i, pl.ds(j * 16, 16)] = zero16
        ncntv[pl.ds(i * 16, 16)] = zero16
        eacc[i, :] = zero16
        ecntv[pl.ds(i * 16, 16)] = zero16
        return c
    lax.fori_loop(0, NUM_GRAPHS, zrow, 0)

    pltpu.sync_copy(b_hbm, btbl)

    n_my = (NODE_NCHUNKS - wid + NW - 1) // NW

    def node_chunk(c, carry):
        base = (wid + c * NW) * NODE_CHUNK
        pltpu.sync_copy(x_hbm.at[pl.ds(base, NODE_CHUNK)], xbuf)
        pltpu.sync_copy(b_hbm.at[pl.ds(base, NODE_CHUNK)], bbuf)
        for g in range(NODE_CHUNK // 16):
            gv = bbuf[pl.ds(g * 16, 16)]
            plsc.addupdate_scatter(ncntv, [gv * 16 + lane], ones16)
            for i in range(16):
                gg = gv[i]
                r = g * 16 + i
                for j in range(D_FEAT // 16):
                    plsc.addupdate(nacc.at[gg, pl.ds(j * 16, 16)],
                                   xbuf[r, pl.ds(j * 16, 16)])
        return carry
    lax.fori_loop(0, n_my, node_chunk, 0)

    def edge_chunk(c, carry):
        base = wid * EDGES_PER_W + c * EDGE_CHUNK
        pltpu.sync_copy(ei_hbm.at[pl.ds(base, EDGE_CHUNK)], srcbuf)
        pltpu.sync_copy(ea_hbm.at[pl.ds(base, EDGE_CHUNK)], eabuf)

        def grp(g, cc):
            sv = srcbuf[pl.ds(g * 16, 16)]
            gv = plsc.load_gather(btbl, [sv])
            plsc.addupdate_scatter(ecntv, [gv * 16 + lane], ones16)
            for i in range(16):
                gg = gv[i]
                plsc.addupdate(eacc.at[gg], eabuf[g * 16 + i])
            return cc
        lax.fori_loop(0, EDGE_CHUNK // 16, grp, 0)
        return carry
    lax.fori_loop(0, EDGE_NCHUNKS, edge_chunk, 0)

    pltpu.sync_copy(nacc, nsum_hbm.at[wid])
    pltpu.sync_copy(ncntv, ncnt_hbm.at[wid])
    pltpu.sync_copy(eacc, esum_hbm.at[wid])
    pltpu.sync_copy(ecntv, ecnt_hbm.at[wid])


_sc_pool = functools.partial(
    pl.kernel,
    out_type=[
        jax.ShapeDtypeStruct((NW, NUM_GRAPHS, D_FEAT), jnp.float32),
        jax.ShapeDtypeStruct((NW, NUM_GRAPHS * 16), jnp.float32),
        jax.ShapeDtypeStruct((NW, NUM_GRAPHS, D_EDGE), jnp.float32),
        jax.ShapeDtypeStruct((NW, NUM_GRAPHS * 16), jnp.float32),
    ],
    mesh=plsc.VectorSubcoreMesh(core_axis_name="c", subcore_axis_name="s"),
    compiler_params=pltpu.CompilerParams(needs_layout_passes=False, use_tc_tiling_on_sc=False),
    scratch_types=[
        pltpu.VMEM((N_NODES,), jnp.int32),
        pltpu.VMEM((NODE_CHUNK, D_FEAT), jnp.float32),
        pltpu.VMEM((NODE_CHUNK,), jnp.int32),
        pltpu.VMEM((EDGE_CHUNK,), jnp.int32),
        pltpu.VMEM((EDGE_CHUNK, D_EDGE), jnp.float32),
        pltpu.VMEM((NUM_GRAPHS, D_FEAT), jnp.float32),
        pltpu.VMEM((NUM_GRAPHS * 16,), jnp.float32),
        pltpu.VMEM((NUM_GRAPHS, D_EDGE), jnp.float32),
        pltpu.VMEM((NUM_GRAPHS * 16,), jnp.float32),
    ],
)(_sc_pool_body)


def _mlp_body(nsum, ncnt, esum, ecnt, w1, b1, w2, b2, out):
    ns = jnp.sum(nsum[...], axis=0)
    nc = jnp.sum(ncnt[...].reshape(NW, NUM_GRAPHS, 16), axis=(0, 2))[:, None]
    es = jnp.sum(esum[...], axis=0)
    ec = jnp.sum(ecnt[...].reshape(NW, NUM_GRAPHS, 16), axis=(0, 2))[:, None]
    nmean = ns / jnp.maximum(nc, 1.0)
    emean = es / jnp.maximum(ec, 1.0)
    w = w1[...]
    h = (jnp.dot(nmean, w[:D_FEAT], preferred_element_type=jnp.float32)
         + jnp.dot(emean, w[D_FEAT:], preferred_element_type=jnp.float32)
         + b1[...])
    h = jnp.maximum(h, 0.0)
    out[...] = jnp.dot(h, w2[...], preferred_element_type=jnp.float32) + b2[...]


def kernel(x, edge_index, edge_attr, batch, W1, b1, W2, b2):
    ei = edge_index[0].astype(jnp.int32)
    b = batch.astype(jnp.int32)
    nsum, ncnt, esum, ecnt = _sc_pool(x, ei, edge_attr, b)
    out = pl.pallas_call(
        _mlp_body,
        out_shape=jax.ShapeDtypeStruct((NUM_GRAPHS, OUT_DIM), jnp.float32),
    )(nsum, ncnt, esum, ecnt, W1, b1.reshape(1, HIDDEN), W2,
      b2.reshape(1, OUT_DIM))
    return out


# speedup vs baseline: 18.0745x; 1.0001x over previous
import functools

import jax
import jax.numpy as jnp
from jax import lax
from jax.experimental import pallas as pl
from jax.experimental.pallas import tpu as pltpu
from jax.experimental.pallas import tpu_sc as plsc

NUM_GRAPHS = 64
N_NODES = 10000
N_EDGES = 640000
D_FEAT = 128
D_EDGE = 16
HIDDEN = 256
OUT_DIM = 128

NW = 32
NODE_CHUNK = 80
NODE_NCHUNKS = N_NODES // NODE_CHUNK
EDGES_PER_W = N_EDGES // NW
EDGE_CHUNK = 2000
EDGE_NCHUNKS = EDGES_PER_W // EDGE_CHUNK


def _sc_pool_body(x_hbm, ei_hbm, ea_hbm, b_hbm,
                  nsum_hbm, ncnt_hbm, esum_hbm, ecnt_hbm,
                  btbl, xbuf, bbuf, srcbuf, eabuf,
                  nacc, ncntv, eacc, ecntv):
    cid = lax.axis_index("c")
    sid = lax.axis_index("s")
    wid = sid * 2 + cid

    zero16 = jnp.zeros((16,), jnp.float32)
    ones16 = jnp.ones((16,), jnp.float32)
    lane = lax.iota(jnp.int32, 16)

    def zrow(i, c):
        for j in range(D_FEAT // 16):
            nacc[i, pl.ds(j * 16, 16)] = zero16
        ncntv[pl.ds(i * 16, 16)] = zero16
        eacc[pl.ds(i * 16, 16)] = zero16
        ecntv[pl.ds(i * 16, 16)] = zero16
        return c
    lax.fori_loop(0, NUM_GRAPHS, zrow, 0)

    pltpu.sync_copy(b_hbm, btbl)

    n_my = (NODE_NCHUNKS - wid + NW - 1) // NW

    def node_chunk(c, carry):
        base = (wid + c * NW) * NODE_CHUNK
        pltpu.sync_copy(x_hbm.at[pl.ds(base, NODE_CHUNK)], xbuf)
        pltpu.sync_copy(b_hbm.at[pl.ds(base, NODE_CHUNK)], bbuf)
        for g in range(NODE_CHUNK // 16):
            gv = bbuf[pl.ds(g * 16, 16)]
            plsc.addupdate_scatter(ncntv, [gv * 16 + lane], ones16)
            for i in range(16):
                gg = gv[i]
                r = g * 16 + i
                for j in range(D_FEAT // 16):
                    plsc.addupdate(nacc.at[gg, pl.ds(j * 16, 16)],
                                   xbuf[r, pl.ds(j * 16, 16)])
        return carry
    lax.fori_loop(0, n_my, node_chunk, 0)

    def edge_chunk(c, carry):
        base = wid * EDGES_PER_W + c * EDGE_CHUNK
        pltpu.sync_copy(ei_hbm.at[pl.ds(base, EDGE_CHUNK)], srcbuf)
        pltpu.sync_copy(ea_hbm.at[pl.ds(base // 8, EDGE_CHUNK // 8)], eabuf)

        def grp(g, cc):
            sv = srcbuf[pl.ds(g * 16, 16)]
            gv = plsc.load_gather(btbl, [sv])
            plsc.addupdate_scatter(ecntv, [gv * 16 + lane], ones16)
            row = g * 2
            for i in range(16):
                gg = gv[i]
                plsc.addupdate(eacc.at[pl.ds(gg * 16, 16)],
                               eabuf[row + (i >> 3), pl.ds((i & 7) * 16, 16)])
            return cc
        lax.fori_loop(0, EDGE_CHUNK // 16, grp, 0)
        return carry
    lax.fori_loop(0, EDGE_NCHUNKS, edge_chunk, 0)

    pltpu.sync_copy(nacc, nsum_hbm.at[wid])
    pltpu.sync_copy(ncntv, ncnt_hbm.at[wid])
    pltpu.sync_copy(eacc, esum_hbm.at[wid])
    pltpu.sync_copy(ecntv, ecnt_hbm.at[wid])


_sc_pool = functools.partial(
    pl.kernel,
    out_type=[
        jax.ShapeDtypeStruct((NW, NUM_GRAPHS, D_FEAT), jnp.float32),
        jax.ShapeDtypeStruct((NW, NUM_GRAPHS * 16), jnp.float32),
        jax.ShapeDtypeStruct((NW, NUM_GRAPHS * D_EDGE), jnp.float32),
        jax.ShapeDtypeStruct((NW, NUM_GRAPHS * 16), jnp.float32),
    ],
    mesh=plsc.VectorSubcoreMesh(core_axis_name="c", subcore_axis_name="s"),
    compiler_params=pltpu.CompilerParams(needs_layout_passes=False, use_tc_tiling_on_sc=False),
    scratch_types=[
        pltpu.VMEM((N_NODES,), jnp.int32),
        pltpu.VMEM((NODE_CHUNK, D_FEAT), jnp.float32),
        pltpu.VMEM((NODE_CHUNK,), jnp.int32),
        pltpu.VMEM((EDGE_CHUNK,), jnp.int32),
        pltpu.VMEM((EDGE_CHUNK // 8, 128), jnp.float32),
        pltpu.VMEM((NUM_GRAPHS, D_FEAT), jnp.float32),
        pltpu.VMEM((NUM_GRAPHS * 16,), jnp.float32),
        pltpu.VMEM((NUM_GRAPHS * D_EDGE,), jnp.float32),
        pltpu.VMEM((NUM_GRAPHS * 16,), jnp.float32),
    ],
)(_sc_pool_body)


def _mlp_body(nsum, ncnt, esum, ecnt, w1, b1, w2, b2, out):
    ns = jnp.sum(nsum[...], axis=0)
    nc = jnp.sum(ncnt[...].reshape(NW, NUM_GRAPHS, 16), axis=(0, 2))[:, None]
    es = jnp.sum(esum[...].reshape(NW, NUM_GRAPHS, D_EDGE), axis=0)
    ec = jnp.sum(ecnt[...].reshape(NW, NUM_GRAPHS, 16), axis=(0, 2))[:, None]
    nmean = ns / jnp.maximum(nc, 1.0)
    emean = es / jnp.maximum(ec, 1.0)
    w = w1[...]
    h = (jnp.dot(nmean, w[:D_FEAT], preferred_element_type=jnp.float32)
         + jnp.dot(emean, w[D_FEAT:], preferred_element_type=jnp.float32)
         + b1[...])
    h = jnp.maximum(h, 0.0)
    out[...] = jnp.dot(h, w2[...], preferred_element_type=jnp.float32) + b2[...]


def kernel(x, edge_index, edge_attr, batch, W1, b1, W2, b2):
    ei = edge_index[0].astype(jnp.int32)
    b = batch.astype(jnp.int32)
    ea = edge_attr.reshape(N_EDGES // 8, 8 * D_EDGE)
    nsum, ncnt, esum, ecnt = _sc_pool(x, ei, ea, b)
    out = pl.pallas_call(
        _mlp_body,
        out_shape=jax.ShapeDtypeStruct((NUM_GRAPHS, OUT_DIM), jnp.float32),
    )(nsum, ncnt, esum, ecnt, W1, b1.reshape(1, HIDDEN), W2,
      b2.reshape(1, OUT_DIM))
    return out


# speedup vs baseline: 28.1164x; 1.5556x over previous
import functools

import jax
import jax.numpy as jnp
from jax import lax
from jax.experimental import pallas as pl
from jax.experimental.pallas import tpu as pltpu
from jax.experimental.pallas import tpu_sc as plsc

NUM_GRAPHS = 64
N_NODES = 10000
N_EDGES = 640000
D_FEAT = 128
D_EDGE = 16
HIDDEN = 256
OUT_DIM = 128

NW = 32
NODE_CHUNK = 80
NODE_NCHUNKS = N_NODES // NODE_CHUNK
EDGE_CHUNK = 2560
EDGE_NCHUNKS = N_EDGES // EDGE_CHUNK


def _sc_pool_body(x_hbm, ei_hbm, ea_hbm, b_hbm,
                  nsum_hbm, ncnt_hbm, esum_hbm, ecnt_hbm,
                  btbl, xbuf, bbuf, srcbuf, eabuf,
                  nacc, ncntv, eacc, ecntv):
    cid = lax.axis_index("c")
    sid = lax.axis_index("s")
    wid = sid * 2 + cid

    zero16 = jnp.zeros((16,), jnp.float32)
    ones16 = jnp.ones((16,), jnp.float32)
    lane = lax.iota(jnp.int32, 16)

    def zrow(i, c):
        for j in range(D_FEAT // 16):
            nacc[i, pl.ds(j * 16, 16)] = zero16
        ncntv[pl.ds(i * 16, 16)] = zero16
        eacc[pl.ds(i * 16, 16)] = zero16
        ecntv[pl.ds(i * 16, 16)] = zero16
        return c
    lax.fori_loop(0, NUM_GRAPHS, zrow, 0)

    pltpu.sync_copy(b_hbm, btbl)

    n_my = (NODE_NCHUNKS - wid + NW - 1) // NW

    def node_chunk(c, carry):
        base = (wid + c * NW) * NODE_CHUNK
        pltpu.sync_copy(x_hbm.at[pl.ds(base, NODE_CHUNK)], xbuf)
        pltpu.sync_copy(b_hbm.at[pl.ds(base, NODE_CHUNK)], bbuf)
        for g in range(NODE_CHUNK // 16):
            gv = bbuf[pl.ds(g * 16, 16)]
            plsc.addupdate_scatter(ncntv, [gv * 16 + lane], ones16)
            for i in range(16):
                gg = gv[i]
                r = g * 16 + i
                for j in range(D_FEAT // 16):
                    plsc.addupdate(nacc.at[gg, pl.ds(j * 16, 16)],
                                   xbuf[r, pl.ds(j * 16, 16)])
        return carry
    lax.fori_loop(0, n_my, node_chunk, 0)

    e_my = (EDGE_NCHUNKS - wid + NW - 1) // NW

    def edge_chunk(c, carry):
        base = (wid + c * NW) * EDGE_CHUNK
        pltpu.sync_copy(ei_hbm.at[pl.ds(base, EDGE_CHUNK)], srcbuf)
        pltpu.sync_copy(ea_hbm.at[:, pl.ds(base, EDGE_CHUNK)], eabuf)

        def grp(g, cc):
            sv = srcbuf[pl.ds(g * 16, 16)]
            gv = plsc.load_gather(btbl, [sv])
            plsc.addupdate_scatter(ecntv, [gv * 16 + lane], ones16)
            idx0 = gv * 16
            for f in range(D_EDGE):
                plsc.addupdate_scatter(eacc, [idx0 + f],
                                       eabuf[f, pl.ds(g * 16, 16)])
            return cc
        lax.fori_loop(0, EDGE_CHUNK // 16, grp, 0)
        return carry
    lax.fori_loop(0, e_my, edge_chunk, 0)

    pltpu.sync_copy(nacc, nsum_hbm.at[wid])
    pltpu.sync_copy(ncntv, ncnt_hbm.at[wid])
    pltpu.sync_copy(eacc, esum_hbm.at[wid])
    pltpu.sync_copy(ecntv, ecnt_hbm.at[wid])


_sc_pool = functools.partial(
    pl.kernel,
    out_type=[
        jax.ShapeDtypeStruct((NW, NUM_GRAPHS, D_FEAT), jnp.float32),
        jax.ShapeDtypeStruct((NW, NUM_GRAPHS * 16), jnp.float32),
        jax.ShapeDtypeStruct((NW, NUM_GRAPHS * D_EDGE), jnp.float32),
        jax.ShapeDtypeStruct((NW, NUM_GRAPHS * 16), jnp.float32),
    ],
    mesh=plsc.VectorSubcoreMesh(core_axis_name="c", subcore_axis_name="s"),
    compiler_params=pltpu.CompilerParams(needs_layout_passes=False,
                                         use_tc_tiling_on_sc=True),
    scratch_types=[
        pltpu.VMEM((N_NODES,), jnp.int32),
        pltpu.VMEM((NODE_CHUNK, D_FEAT), jnp.float32),
        pltpu.VMEM((NODE_CHUNK,), jnp.int32),
        pltpu.VMEM((EDGE_CHUNK,), jnp.int32),
        pltpu.VMEM((D_EDGE, EDGE_CHUNK), jnp.float32),
        pltpu.VMEM((NUM_GRAPHS, D_FEAT), jnp.float32),
        pltpu.VMEM((NUM_GRAPHS * 16,), jnp.float32),
        pltpu.VMEM((NUM_GRAPHS * D_EDGE,), jnp.float32),
        pltpu.VMEM((NUM_GRAPHS * 16,), jnp.float32),
    ],
)(_sc_pool_body)


def _mlp_body(nsum, ncnt, esum, ecnt, w1, b1, w2, b2, out):
    ns = jnp.sum(nsum[...], axis=0)
    nc = jnp.sum(ncnt[...].reshape(NW, NUM_GRAPHS, 16), axis=(0, 2))[:, None]
    es = jnp.sum(esum[...].reshape(NW, NUM_GRAPHS, D_EDGE), axis=0)
    ec = jnp.sum(ecnt[...].reshape(NW, NUM_GRAPHS, 16), axis=(0, 2))[:, None]
    nmean = ns / jnp.maximum(nc, 1.0)
    emean = es / jnp.maximum(ec, 1.0)
    w = w1[...]
    h = (jnp.dot(nmean, w[:D_FEAT], preferred_element_type=jnp.float32)
         + jnp.dot(emean, w[D_FEAT:], preferred_element_type=jnp.float32)
         + b1[...])
    h = jnp.maximum(h, 0.0)
    out[...] = jnp.dot(h, w2[...], preferred_element_type=jnp.float32) + b2[...]


def kernel(x, edge_index, edge_attr, batch, W1, b1, W2, b2):
    ei = edge_index[0].astype(jnp.int32)
    b = batch.astype(jnp.int32)
    ea_t = edge_attr.T
    nsum, ncnt, esum, ecnt = _sc_pool(x, ei, ea_t, b)
    out = pl.pallas_call(
        _mlp_body,
        out_shape=jax.ShapeDtypeStruct((NUM_GRAPHS, OUT_DIM), jnp.float32),
    )(nsum, ncnt, esum, ecnt, W1, b1.reshape(1, HIDDEN), W2,
      b2.reshape(1, OUT_DIM))
    return out


# speedup vs baseline: 40.2289x; 1.4308x over previous
import functools

import jax
import jax.numpy as jnp
from jax import lax
from jax.experimental import pallas as pl
from jax.experimental.pallas import tpu as pltpu
from jax.experimental.pallas import tpu_sc as plsc

NUM_GRAPHS = 64
N_NODES = 10000
N_EDGES = 640000
D_FEAT = 128
D_EDGE = 16
HIDDEN = 256
OUT_DIM = 128

NW = 32
NODE_CHUNK = 80
NODE_NCHUNKS = N_NODES // NODE_CHUNK
EDGE_CHUNK = 2560
EDGE_NCHUNKS = N_EDGES // EDGE_CHUNK


def _sc_pool_body(x_hbm, ei_hbm, ea_hbm, b_hbm,
                  nsum_hbm, ncnt_hbm, esum_hbm, ecnt_hbm,
                  btbl, xbuf, bbuf, srcbuf, eabuf,
                  nacc, ncntv, ecntv, *eaccs):
    cid = lax.axis_index("c")
    sid = lax.axis_index("s")
    wid = sid * 2 + cid

    zero16 = jnp.zeros((16,), jnp.float32)
    ones16 = jnp.ones((16,), jnp.float32)
    lane = lax.iota(jnp.int32, 16)

    def zrow(i, c):
        for j in range(D_FEAT // 16):
            nacc[i, pl.ds(j * 16, 16)] = zero16
        ncntv[pl.ds(i * 16, 16)] = zero16
        ecntv[pl.ds(i * 16, 16)] = zero16
        return c
    lax.fori_loop(0, NUM_GRAPHS, zrow, 0)
    for f in range(D_EDGE):
        for i in range(NUM_GRAPHS // 16):
            eaccs[f][pl.ds(i * 16, 16)] = zero16

    pltpu.sync_copy(b_hbm, btbl)

    n_my = (NODE_NCHUNKS - wid + NW - 1) // NW

    def node_chunk(c, carry):
        base = (wid + c * NW) * NODE_CHUNK
        pltpu.sync_copy(x_hbm.at[pl.ds(base, NODE_CHUNK)], xbuf)
        pltpu.sync_copy(b_hbm.at[pl.ds(base, NODE_CHUNK)], bbuf)
        for g in range(NODE_CHUNK // 16):
            gv = bbuf[pl.ds(g * 16, 16)]
            plsc.addupdate_scatter(ncntv, [gv * 16 + lane], ones16)
            for i in range(16):
                gg = gv[i]
                r = g * 16 + i
                for j in range(D_FEAT // 16):
                    plsc.addupdate(nacc.at[gg, pl.ds(j * 16, 16)],
                                   xbuf[r, pl.ds(j * 16, 16)])
        return carry
    lax.fori_loop(0, n_my, node_chunk, 0)

    e_my = (EDGE_NCHUNKS - wid + NW - 1) // NW

    def edge_chunk(c, carry):
        base = (wid + c * NW) * EDGE_CHUNK
        pltpu.sync_copy(ei_hbm.at[pl.ds(base, EDGE_CHUNK)], srcbuf)
        pltpu.sync_copy(ea_hbm.at[:, pl.ds(base, EDGE_CHUNK)], eabuf)

        def grp(g, cc):
            sv = srcbuf[pl.ds(g * 16, 16)]
            gv = plsc.load_gather(btbl, [sv])
            plsc.addupdate_scatter(ecntv, [gv * 16 + lane], ones16)
            for f in range(D_EDGE):
                plsc.addupdate_scatter(eaccs[f], [gv],
                                       eabuf[f, pl.ds(g * 16, 16)])
            return cc
        lax.fori_loop(0, EDGE_CHUNK // 16, grp, 0)
        return carry
    lax.fori_loop(0, e_my, edge_chunk, 0)

    pltpu.sync_copy(nacc, nsum_hbm.at[wid])
    pltpu.sync_copy(ncntv, ncnt_hbm.at[wid])
    for f in range(D_EDGE):
        pltpu.sync_copy(eaccs[f], esum_hbm.at[wid, f])
    pltpu.sync_copy(ecntv, ecnt_hbm.at[wid])


_sc_pool = functools.partial(
    pl.kernel,
    out_type=[
        jax.ShapeDtypeStruct((NW, NUM_GRAPHS, D_FEAT), jnp.float32),
        jax.ShapeDtypeStruct((NW, NUM_GRAPHS * 16), jnp.float32),
        jax.ShapeDtypeStruct((NW, D_EDGE, NUM_GRAPHS), jnp.float32),
        jax.ShapeDtypeStruct((NW, NUM_GRAPHS * 16), jnp.float32),
    ],
    mesh=plsc.VectorSubcoreMesh(core_axis_name="c", subcore_axis_name="s"),
    compiler_params=pltpu.CompilerParams(needs_layout_passes=False,
                                         use_tc_tiling_on_sc=True),
    scratch_types=[
        pltpu.VMEM((N_NODES,), jnp.int32),
        pltpu.VMEM((NODE_CHUNK, D_FEAT), jnp.float32),
        pltpu.VMEM((NODE_CHUNK,), jnp.int32),
        pltpu.VMEM((EDGE_CHUNK,), jnp.int32),
        pltpu.VMEM((D_EDGE, EDGE_CHUNK), jnp.float32),
        pltpu.VMEM((NUM_GRAPHS, D_FEAT), jnp.float32),
        pltpu.VMEM((NUM_GRAPHS * 16,), jnp.float32),
        pltpu.VMEM((NUM_GRAPHS * 16,), jnp.float32),
    ] + [pltpu.VMEM((NUM_GRAPHS,), jnp.float32) for _ in range(D_EDGE)],
)(_sc_pool_body)


def _mlp_body(nsum, ncnt, esum, ecnt, w1, b1, w2, b2, out):
    ns = jnp.sum(nsum[...], axis=0)
    nc = jnp.sum(ncnt[...].reshape(NW, NUM_GRAPHS, 16), axis=(0, 2))[:, None]
    es = jnp.sum(esum[...], axis=0).T
    ec = jnp.sum(ecnt[...].reshape(NW, NUM_GRAPHS, 16), axis=(0, 2))[:, None]
    nmean = ns / jnp.maximum(nc, 1.0)
    emean = es / jnp.maximum(ec, 1.0)
    w = w1[...]
    h = (jnp.dot(nmean, w[:D_FEAT], preferred_element_type=jnp.float32)
         + jnp.dot(emean, w[D_FEAT:], preferred_element_type=jnp.float32)
         + b1[...])
    h = jnp.maximum(h, 0.0)
    out[...] = jnp.dot(h, w2[...], preferred_element_type=jnp.float32) + b2[...]


def kernel(x, edge_index, edge_attr, batch, W1, b1, W2, b2):
    ei = edge_index[0].astype(jnp.int32)
    b = batch.astype(jnp.int32)
    ea_t = edge_attr.T
    nsum, ncnt, esum, ecnt = _sc_pool(x, ei, ea_t, b)
    out = pl.pallas_call(
        _mlp_body,
        out_shape=jax.ShapeDtypeStruct((NUM_GRAPHS, OUT_DIM), jnp.float32),
    )(nsum, ncnt, esum, ecnt, W1, b1.reshape(1, HIDDEN), W2,
      b2.reshape(1, OUT_DIM))
    return out


# speedup vs baseline: 46.3226x; 1.1515x over previous
import functools

import jax
import jax.numpy as jnp
from jax import lax
from jax.experimental import pallas as pl
from jax.experimental.pallas import tpu as pltpu
from jax.experimental.pallas import tpu_sc as plsc

NUM_GRAPHS = 64
N_NODES = 10000
N_EDGES = 640000
D_FEAT = 128
D_EDGE = 16
HIDDEN = 256
OUT_DIM = 128

NW = 32
NODE_CHUNK = 80
NODE_NCHUNKS = N_NODES // NODE_CHUNK
NODE_SLOTS = 4
EDGE_CHUNK = 1280
EDGE_NCHUNKS = N_EDGES // EDGE_CHUNK
EDGE_SLOTS = 16


def _sc_pool_body(x_hbm, ei_hbm, ea_hbm, b_hbm,
                  nsum_hbm, ncnt_hbm, esum_hbm, ecnt_hbm,
                  btbl, nacc, ncntv, ecntv,
                  xbufs, bbufs, srcbufs, eabufs,
                  bsem, xsems, bsems, ssems, easems, *eaccs):
    cid = lax.axis_index("c")
    sid = lax.axis_index("s")
    wid = sid * 2 + cid

    zero16 = jnp.zeros((16,), jnp.float32)
    ones16 = jnp.ones((16,), jnp.float32)
    lane = lax.iota(jnp.int32, 16)

    btbl_copy = pltpu.async_copy(b_hbm, btbl, bsem)

    def zrow(i, c):
        for j in range(D_FEAT // 16):
            nacc[i, pl.ds(j * 16, 16)] = zero16
        ncntv[pl.ds(i * 16, 16)] = zero16
        ecntv[pl.ds(i * 16, 16)] = zero16
        return c
    lax.fori_loop(0, NUM_GRAPHS, zrow, 0)
    for f in range(D_EDGE):
        for i in range(NUM_GRAPHS // 16):
            eaccs[f][pl.ds(i * 16, 16)] = zero16

    def n_start(b, s):
        ch = wid + s * NW
        base = jnp.where(ch < NODE_NCHUNKS, ch, 0) * NODE_CHUNK
        pltpu.async_copy(x_hbm.at[pl.ds(base, NODE_CHUNK)], xbufs[b], xsems[b])
        pltpu.async_copy(b_hbm.at[pl.ds(base, NODE_CHUNK)], bbufs[b], bsems[b])

    def n_wait(b):
        pltpu.make_async_copy(x_hbm.at[pl.ds(0, NODE_CHUNK)], xbufs[b],
                              xsems[b]).wait()
        pltpu.make_async_copy(b_hbm.at[pl.ds(0, NODE_CHUNK)], bbufs[b],
                              bsems[b]).wait()

    def n_proc(b, s):
        ch = wid + s * NW

        @pl.when(ch < NODE_NCHUNKS)
        def _():
            xbuf, bbuf = xbufs[b], bbufs[b]
            for g in range(NODE_CHUNK // 16):
                gv = bbuf[pl.ds(g * 16, 16)]
                plsc.addupdate_scatter(ncntv, [gv * 16 + lane], ones16)
                for i in range(16):
                    gg = gv[i]
                    r = g * 16 + i
                    for j in range(D_FEAT // 16):
                        plsc.addupdate(nacc.at[gg, pl.ds(j * 16, 16)],
                                       xbuf[r, pl.ds(j * 16, 16)])

    n_start(0, 0)

    def n_pair(j, carry):
        s0 = j * 2
        n_start(1, s0 + 1)
        n_wait(0)
        n_proc(0, s0)
        n_start(0, s0 + 2)
        n_wait(1)
        n_proc(1, s0 + 1)
        return carry
    lax.fori_loop(0, NODE_SLOTS // 2, n_pair, 0)
    n_wait(0)

    btbl_copy.wait()

    def e_start(b, s):
        ch = wid + s * NW
        base = jnp.where(ch < EDGE_NCHUNKS, ch, 0) * EDGE_CHUNK
        pltpu.async_copy(ei_hbm.at[pl.ds(base, EDGE_CHUNK)], srcbufs[b],
                         ssems[b])
        pltpu.async_copy(ea_hbm.at[:, pl.ds(base, EDGE_CHUNK)], eabufs[b],
                         easems[b])

    def e_wait(b):
        pltpu.make_async_copy(ei_hbm.at[pl.ds(0, EDGE_CHUNK)], srcbufs[b],
                              ssems[b]).wait()
        pltpu.make_async_copy(ea_hbm.at[:, pl.ds(0, EDGE_CHUNK)], eabufs[b],
                              easems[b]).wait()

    def e_proc(b, s):
        ch = wid + s * NW
        ngrp = jnp.where(ch < EDGE_NCHUNKS, EDGE_CHUNK // 16, 0)
        srcbuf, eabuf = srcbufs[b], eabufs[b]

        def grp(g, cc):
            sv = srcbuf[pl.ds(g * 16, 16)]
            gv = plsc.load_gather(btbl, [sv])
            plsc.addupdate_scatter(ecntv, [gv * 16 + lane], ones16)
            for f in range(D_EDGE):
                plsc.addupdate_scatter(eaccs[f], [gv],
                                       eabuf[f, pl.ds(g * 16, 16)])
            return cc
        lax.fori_loop(0, ngrp, grp, 0)

    e_start(0, 0)

    def e_pair(j, carry):
        s0 = j * 2
        e_start(1, s0 + 1)
        e_wait(0)
        e_proc(0, s0)
        e_start(0, s0 + 2)
        e_wait(1)
        e_proc(1, s0 + 1)
        return carry
    lax.fori_loop(0, EDGE_SLOTS // 2, e_pair, 0)
    e_wait(0)

    pltpu.sync_copy(nacc, nsum_hbm.at[wid])
    pltpu.sync_copy(ncntv, ncnt_hbm.at[wid])
    for f in range(D_EDGE):
        pltpu.sync_copy(eaccs[f], esum_hbm.at[wid, f])
    pltpu.sync_copy(ecntv, ecnt_hbm.at[wid])


_sc_pool = functools.partial(
    pl.kernel,
    out_type=[
        jax.ShapeDtypeStruct((NW, NUM_GRAPHS, D_FEAT), jnp.float32),
        jax.ShapeDtypeStruct((NW, NUM_GRAPHS * 16), jnp.float32),
        jax.ShapeDtypeStruct((NW, D_EDGE, NUM_GRAPHS), jnp.float32),
        jax.ShapeDtypeStruct((NW, NUM_GRAPHS * 16), jnp.float32),
    ],
    mesh=plsc.VectorSubcoreMesh(core_axis_name="c", subcore_axis_name="s"),
    compiler_params=pltpu.CompilerParams(needs_layout_passes=False,
                                         use_tc_tiling_on_sc=True),
    scratch_types=[
        pltpu.VMEM((N_NODES,), jnp.int32),
        pltpu.VMEM((NUM_GRAPHS, D_FEAT), jnp.float32),
        pltpu.VMEM((NUM_GRAPHS * 16,), jnp.float32),
        pltpu.VMEM((NUM_GRAPHS * 16,), jnp.float32),
        [pltpu.VMEM((NODE_CHUNK, D_FEAT), jnp.float32) for _ in range(2)],
        [pltpu.VMEM((NODE_CHUNK,), jnp.int32) for _ in range(2)],
        [pltpu.VMEM((EDGE_CHUNK,), jnp.int32) for _ in range(2)],
        [pltpu.VMEM((D_EDGE, EDGE_CHUNK), jnp.float32) for _ in range(2)],
        pltpu.SemaphoreType.DMA,
        [pltpu.SemaphoreType.DMA for _ in range(2)],
        [pltpu.SemaphoreType.DMA for _ in range(2)],
        [pltpu.SemaphoreType.DMA for _ in range(2)],
        [pltpu.SemaphoreType.DMA for _ in range(2)],
    ] + [pltpu.VMEM((NUM_GRAPHS,), jnp.float32) for _ in range(D_EDGE)],
)(_sc_pool_body)


def _mlp_body(nsum, ncnt, esum, ecnt, w1, b1, w2, b2, out):
    ns = jnp.sum(nsum[...], axis=0)
    nc = jnp.sum(ncnt[...].reshape(NW, NUM_GRAPHS, 16), axis=(0, 2))[:, None]
    es = jnp.sum(esum[...], axis=0).T
    ec = jnp.sum(ecnt[...].reshape(NW, NUM_GRAPHS, 16), axis=(0, 2))[:, None]
    nmean = ns / jnp.maximum(nc, 1.0)
    emean = es / jnp.maximum(ec, 1.0)
    w = w1[...]
    h = (jnp.dot(nmean, w[:D_FEAT], preferred_element_type=jnp.float32)
         + jnp.dot(emean, w[D_FEAT:], preferred_element_type=jnp.float32)
         + b1[...])
    h = jnp.maximum(h, 0.0)
    out[...] = jnp.dot(h, w2[...], preferred_element_type=jnp.float32) + b2[...]


def kernel(x, edge_index, edge_attr, batch, W1, b1, W2, b2):
    ei = edge_index[0].astype(jnp.int32)
    b = batch.astype(jnp.int32)
    ea_t = edge_attr.T
    nsum, ncnt, esum, ecnt = _sc_pool(x, ei, ea_t, b)
    out = pl.pallas_call(
        _mlp_body,
        out_shape=jax.ShapeDtypeStruct((NUM_GRAPHS, OUT_DIM), jnp.float32),
    )(nsum, ncnt, esum, ecnt, W1, b1.reshape(1, HIDDEN), W2,
      b2.reshape(1, OUT_DIM))
    return out


# speedup vs baseline: 46.8809x; 1.0121x over previous
import functools

import jax
import jax.numpy as jnp
from jax import lax
from jax.experimental import pallas as pl
from jax.experimental.pallas import tpu as pltpu
from jax.experimental.pallas import tpu_sc as plsc

NUM_GRAPHS = 64
N_NODES = 10000
N_EDGES = 640000
D_FEAT = 128
D_EDGE = 16
HIDDEN = 256
OUT_DIM = 128

NW = 32
NODE_CHUNK = 80
NODE_NCHUNKS = N_NODES // NODE_CHUNK
NODE_SLOTS = 4
EDGE_CHUNK = 1280
EDGE_NCHUNKS = N_EDGES // EDGE_CHUNK
EDGE_SLOTS = 16


def _sc_pool_body(x_hbm, ei_hbm, ea_hbm, b_hbm,
                  nsum_hbm, ncnt_hbm, esum_hbm,
                  btbl, nacc, ncntv,
                  xbufs, bbufs, srcbufs, eabufs,
                  bsem, xsems, bsems, ssems, easems, osem, *eaccs):
    cid = lax.axis_index("c")
    sid = lax.axis_index("s")
    wid = sid * 2 + cid

    zero16 = jnp.zeros((16,), jnp.float32)
    ones16 = jnp.ones((16,), jnp.float32)
    lane = lax.iota(jnp.int32, 16)

    btbl_copy = pltpu.async_copy(b_hbm, btbl, bsem)

    def zrow(i, c):
        for j in range(D_FEAT // 16):
            nacc[i, pl.ds(j * 16, 16)] = zero16
        ncntv[pl.ds(i * 16, 16)] = zero16
        return c
    lax.fori_loop(0, NUM_GRAPHS, zrow, 0)
    for f in range(D_EDGE + 1):
        for i in range(NUM_GRAPHS // 16):
            eaccs[f][pl.ds(i * 16, 16)] = zero16

    def n_start(b, s):
        ch = wid + s * NW
        base = jnp.where(ch < NODE_NCHUNKS, ch, 0) * NODE_CHUNK
        pltpu.async_copy(x_hbm.at[pl.ds(base, NODE_CHUNK)], xbufs[b], xsems[b])
        pltpu.async_copy(b_hbm.at[pl.ds(base, NODE_CHUNK)], bbufs[b], bsems[b])

    def n_wait(b):
        pltpu.make_async_copy(x_hbm.at[pl.ds(0, NODE_CHUNK)], xbufs[b],
                              xsems[b]).wait()
        pltpu.make_async_copy(b_hbm.at[pl.ds(0, NODE_CHUNK)], bbufs[b],
                              bsems[b]).wait()

    def n_proc(b, s):
        ch = wid + s * NW

        @pl.when(ch < NODE_NCHUNKS)
        def _():
            xbuf, bbuf = xbufs[b], bbufs[b]
            for g in range(NODE_CHUNK // 16):
                gv = bbuf[pl.ds(g * 16, 16)]
                plsc.addupdate_scatter(ncntv, [gv * 16 + lane], ones16)
                for i in range(16):
                    gg = gv[i]
                    r = g * 16 + i
                    for j in range(D_FEAT // 16):
                        plsc.addupdate(nacc.at[gg, pl.ds(j * 16, 16)],
                                       xbuf[r, pl.ds(j * 16, 16)])

    n_start(0, 0)

    def n_pair(j, carry):
        s0 = j * 2
        n_start(1, s0 + 1)
        n_wait(0)
        n_proc(0, s0)
        n_start(0, s0 + 2)
        n_wait(1)
        n_proc(1, s0 + 1)
        return carry
    lax.fori_loop(0, NODE_SLOTS // 2, n_pair, 0)
    n_wait(0)

    btbl_copy.wait()

    def e_start(b, s):
        ch = wid + s * NW
        base = jnp.where(ch < EDGE_NCHUNKS, ch, 0) * EDGE_CHUNK
        pltpu.async_copy(ei_hbm.at[pl.ds(base, EDGE_CHUNK)], srcbufs[b],
                         ssems[b])
        pltpu.async_copy(ea_hbm.at[:, pl.ds(base, EDGE_CHUNK)], eabufs[b],
                         easems[b])

    def e_wait(b):
        pltpu.make_async_copy(ei_hbm.at[pl.ds(0, EDGE_CHUNK)], srcbufs[b],
                              ssems[b]).wait()
        pltpu.make_async_copy(ea_hbm.at[:, pl.ds(0, EDGE_CHUNK)], eabufs[b],
                              easems[b]).wait()

    def e_proc(b, s):
        ch = wid + s * NW
        srcbuf, eabuf = srcbufs[b], eabufs[b]

        @pl.when(ch < EDGE_NCHUNKS)
        def _():
            def grp(g, cc):
                sv = srcbuf[pl.ds(g * 16, 16)]
                gv = plsc.load_gather(btbl, [sv])
                plsc.addupdate_scatter(eaccs[D_EDGE], [gv], ones16)
                for f in range(D_EDGE):
                    plsc.addupdate_scatter(eaccs[f], [gv],
                                           eabuf[f, pl.ds(g * 16, 16)])
                return cc
            lax.fori_loop(0, EDGE_CHUNK // 16, grp, 0, unroll=2)

    e_start(0, 0)

    def e_pair(j, carry):
        s0 = j * 2
        e_start(1, s0 + 1)
        e_wait(0)
        e_proc(0, s0)
        e_start(0, s0 + 2)
        e_wait(1)
        e_proc(1, s0 + 1)
        return carry
    lax.fori_loop(0, EDGE_SLOTS // 2, e_pair, 0)
    e_wait(0)

    pltpu.async_copy(nacc, nsum_hbm.at[wid], osem)
    pltpu.async_copy(ncntv, ncnt_hbm.at[wid], osem)
    for f in range(D_EDGE + 1):
        pltpu.async_copy(eaccs[f], esum_hbm.at[wid * (D_EDGE + 1) + f], osem)
    pltpu.make_async_copy(nacc, nsum_hbm.at[wid], osem).wait()
    pltpu.make_async_copy(ncntv, ncnt_hbm.at[wid], osem).wait()
    for f in range(D_EDGE + 1):
        pltpu.make_async_copy(eaccs[f],
                              esum_hbm.at[wid * (D_EDGE + 1) + f], osem).wait()


_sc_pool = functools.partial(
    pl.kernel,
    out_type=[
        jax.ShapeDtypeStruct((NW, NUM_GRAPHS, D_FEAT), jnp.float32),
        jax.ShapeDtypeStruct((NW, NUM_GRAPHS * 16), jnp.float32),
        jax.ShapeDtypeStruct((NW * (D_EDGE + 1), NUM_GRAPHS), jnp.float32),
    ],
    mesh=plsc.VectorSubcoreMesh(core_axis_name="c", subcore_axis_name="s"),
    compiler_params=pltpu.CompilerParams(needs_layout_passes=False,
                                         use_tc_tiling_on_sc=True),
    scratch_types=[
        pltpu.VMEM((N_NODES,), jnp.int32),
        pltpu.VMEM((NUM_GRAPHS, D_FEAT), jnp.float32),
        pltpu.VMEM((NUM_GRAPHS * 16,), jnp.float32),
        [pltpu.VMEM((NODE_CHUNK, D_FEAT), jnp.float32) for _ in range(2)],
        [pltpu.VMEM((NODE_CHUNK,), jnp.int32) for _ in range(2)],
        [pltpu.VMEM((EDGE_CHUNK,), jnp.int32) for _ in range(2)],
        [pltpu.VMEM((D_EDGE, EDGE_CHUNK), jnp.float32) for _ in range(2)],
        pltpu.SemaphoreType.DMA,
        [pltpu.SemaphoreType.DMA for _ in range(2)],
        [pltpu.SemaphoreType.DMA for _ in range(2)],
        [pltpu.SemaphoreType.DMA for _ in range(2)],
        [pltpu.SemaphoreType.DMA for _ in range(2)],
        pltpu.SemaphoreType.DMA,
    ] + [pltpu.VMEM((NUM_GRAPHS,), jnp.float32) for _ in range(D_EDGE + 1)],
)(_sc_pool_body)


def _mlp_body(nsum, ncnt, esum, w1, b1, w2, b2, out):
    ns = jnp.sum(nsum[...], axis=0)
    nc = jnp.sum(ncnt[...].reshape(NW, NUM_GRAPHS, 16), axis=(0, 2))[:, None]
    est = jnp.sum(esum[...].reshape(NW, D_EDGE + 1, NUM_GRAPHS), axis=0)
    es = est[:D_EDGE].T
    ec = est[D_EDGE][:, None]
    nmean = ns / jnp.maximum(nc, 1.0)
    emean = es / jnp.maximum(ec, 1.0)
    w = w1[...]
    h = (jnp.dot(nmean, w[:D_FEAT], preferred_element_type=jnp.float32)
         + jnp.dot(emean, w[D_FEAT:], preferred_element_type=jnp.float32)
         + b1[...])
    h = jnp.maximum(h, 0.0)
    out[...] = jnp.dot(h, w2[...], preferred_element_type=jnp.float32) + b2[...]


def kernel(x, edge_index, edge_attr, batch, W1, b1, W2, b2):
    ei = edge_index[0].astype(jnp.int32)
    b = batch.astype(jnp.int32)
    ea_t = edge_attr.T
    nsum, ncnt, esum = _sc_pool(x, ei, ea_t, b)
    out = pl.pallas_call(
        _mlp_body,
        out_shape=jax.ShapeDtypeStruct((NUM_GRAPHS, OUT_DIM), jnp.float32),
    )(nsum, ncnt, esum, W1, b1.reshape(1, HIDDEN), W2,
      b2.reshape(1, OUT_DIM))
    return out


# speedup vs baseline: 47.5330x; 1.0139x over previous
import functools

import jax
import jax.numpy as jnp
from jax import lax
from jax.experimental import pallas as pl
from jax.experimental.pallas import tpu as pltpu
from jax.experimental.pallas import tpu_sc as plsc

NUM_GRAPHS = 64
N_NODES = 10000
N_EDGES = 640000
D_FEAT = 128
D_EDGE = 16
HIDDEN = 256
OUT_DIM = 128

NW = 32
NODE_CHUNK = 80
NODE_NCHUNKS = N_NODES // NODE_CHUNK
NODE_SLOTS = 4
EDGE_CHUNK = 1280
EDGE_NCHUNKS = N_EDGES // EDGE_CHUNK
EDGE_SLOTS = 16


def _sc_pool_body(x_hbm, ei_hbm, ea_hbm, b_hbm,
                  nsum_hbm, ncnt_hbm, esum_hbm,
                  btbl, nacc, ncntv,
                  xbufs, bbufs, srcbufs, eabufs,
                  bsem, xsems, bsems, ssems, easems, osem, *eaccs):
    cid = lax.axis_index("c")
    sid = lax.axis_index("s")
    wid = sid * 2 + cid

    zero16 = jnp.zeros((16,), jnp.float32)
    ones16 = jnp.ones((16,), jnp.float32)
    lane = lax.iota(jnp.int32, 16)
    lane8 = lax.iota(jnp.int32, 16) & 7

    btbl_copy = pltpu.async_copy(b_hbm, btbl, bsem)

    def zrow(i, c):
        for j in range(D_FEAT // 16):
            nacc[i, pl.ds(j * 16, 16)] = zero16
        ncntv[pl.ds(i * 16, 16)] = zero16
        return c
    lax.fori_loop(0, NUM_GRAPHS, zrow, 0)
    for f in range(D_EDGE + 1):
        for i in range(NUM_GRAPHS * 8 // 16):
            eaccs[f][pl.ds(i * 16, 16)] = zero16

    def n_start(b, s):
        ch = wid + s * NW
        base = jnp.where(ch < NODE_NCHUNKS, ch, 0) * NODE_CHUNK
        pltpu.async_copy(x_hbm.at[pl.ds(base, NODE_CHUNK)], xbufs[b], xsems[b])
        pltpu.async_copy(b_hbm.at[pl.ds(base, NODE_CHUNK)], bbufs[b], bsems[b])

    def n_wait(b):
        pltpu.make_async_copy(x_hbm.at[pl.ds(0, NODE_CHUNK)], xbufs[b],
                              xsems[b]).wait()
        pltpu.make_async_copy(b_hbm.at[pl.ds(0, NODE_CHUNK)], bbufs[b],
                              bsems[b]).wait()

    def n_proc(b, s):
        ch = wid + s * NW

        @pl.when(ch < NODE_NCHUNKS)
        def _():
            xbuf, bbuf = xbufs[b], bbufs[b]
            for g in range(NODE_CHUNK // 16):
                gv = bbuf[pl.ds(g * 16, 16)]
                plsc.addupdate_scatter(ncntv, [gv * 16 + lane], ones16)
                for i in range(16):
                    gg = gv[i]
                    r = g * 16 + i
                    for j in range(D_FEAT // 16):
                        plsc.addupdate(nacc.at[gg, pl.ds(j * 16, 16)],
                                       xbuf[r, pl.ds(j * 16, 16)])

    n_start(0, 0)

    def n_pair(j, carry):
        s0 = j * 2
        n_start(1, s0 + 1)
        n_wait(0)
        n_proc(0, s0)
        n_start(0, s0 + 2)
        n_wait(1)
        n_proc(1, s0 + 1)
        return carry
    lax.fori_loop(0, NODE_SLOTS // 2, n_pair, 0)
    n_wait(0)

    btbl_copy.wait()

    def e_start(b, s):
        ch = wid + s * NW
        base = jnp.where(ch < EDGE_NCHUNKS, ch, 0) * EDGE_CHUNK
        pltpu.async_copy(ei_hbm.at[pl.ds(base, EDGE_CHUNK)], srcbufs[b],
                         ssems[b])
        pltpu.async_copy(ea_hbm.at[:, pl.ds(base, EDGE_CHUNK)], eabufs[b],
                         easems[b])

    def e_wait(b):
        pltpu.make_async_copy(ei_hbm.at[pl.ds(0, EDGE_CHUNK)], srcbufs[b],
                              ssems[b]).wait()
        pltpu.make_async_copy(ea_hbm.at[:, pl.ds(0, EDGE_CHUNK)], eabufs[b],
                              easems[b]).wait()

    def e_proc(b, s):
        ch = wid + s * NW
        srcbuf, eabuf = srcbufs[b], eabufs[b]

        @pl.when(ch < EDGE_NCHUNKS)
        def _():
            def grp(g, cc):
                sv = srcbuf[pl.ds(g * 16, 16)]
                gv = plsc.load_gather(btbl, [sv]) * 8 + lane8
                plsc.addupdate_scatter(eaccs[D_EDGE], [gv], ones16)
                for f in range(D_EDGE):
                    plsc.addupdate_scatter(eaccs[f], [gv],
                                           eabuf[f, pl.ds(g * 16, 16)])
                return cc
            lax.fori_loop(0, EDGE_CHUNK // 16, grp, 0, unroll=2)

    e_start(0, 0)

    def e_pair(j, carry):
        s0 = j * 2
        e_start(1, s0 + 1)
        e_wait(0)
        e_proc(0, s0)
        e_start(0, s0 + 2)
        e_wait(1)
        e_proc(1, s0 + 1)
        return carry
    lax.fori_loop(0, EDGE_SLOTS // 2, e_pair, 0)
    e_wait(0)

    pltpu.async_copy(nacc, nsum_hbm.at[wid], osem)
    pltpu.async_copy(ncntv, ncnt_hbm.at[wid], osem)
    for f in range(D_EDGE + 1):
        pltpu.async_copy(eaccs[f], esum_hbm.at[wid * (D_EDGE + 1) + f], osem)
    pltpu.make_async_copy(nacc, nsum_hbm.at[wid], osem).wait()
    pltpu.make_async_copy(ncntv, ncnt_hbm.at[wid], osem).wait()
    for f in range(D_EDGE + 1):
        pltpu.make_async_copy(eaccs[f],
                              esum_hbm.at[wid * (D_EDGE + 1) + f], osem).wait()


_sc_pool = functools.partial(
    pl.kernel,
    out_type=[
        jax.ShapeDtypeStruct((NW, NUM_GRAPHS, D_FEAT), jnp.float32),
        jax.ShapeDtypeStruct((NW, NUM_GRAPHS * 16), jnp.float32),
        jax.ShapeDtypeStruct((NW * (D_EDGE + 1), NUM_GRAPHS * 8), jnp.float32),
    ],
    mesh=plsc.VectorSubcoreMesh(core_axis_name="c", subcore_axis_name="s"),
    compiler_params=pltpu.CompilerParams(needs_layout_passes=False,
                                         use_tc_tiling_on_sc=True),
    scratch_types=[
        pltpu.VMEM((N_NODES,), jnp.int32),
        pltpu.VMEM((NUM_GRAPHS, D_FEAT), jnp.float32),
        pltpu.VMEM((NUM_GRAPHS * 16,), jnp.float32),
        [pltpu.VMEM((NODE_CHUNK, D_FEAT), jnp.float32) for _ in range(2)],
        [pltpu.VMEM((NODE_CHUNK,), jnp.int32) for _ in range(2)],
        [pltpu.VMEM((EDGE_CHUNK,), jnp.int32) for _ in range(2)],
        [pltpu.VMEM((D_EDGE, EDGE_CHUNK), jnp.float32) for _ in range(2)],
        pltpu.SemaphoreType.DMA,
        [pltpu.SemaphoreType.DMA for _ in range(2)],
        [pltpu.SemaphoreType.DMA for _ in range(2)],
        [pltpu.SemaphoreType.DMA for _ in range(2)],
        [pltpu.SemaphoreType.DMA for _ in range(2)],
        pltpu.SemaphoreType.DMA,
    ] + [pltpu.VMEM((NUM_GRAPHS * 8,), jnp.float32) for _ in range(D_EDGE + 1)],
)(_sc_pool_body)


def _mlp_body(nsum, ncnt, esum, w1, b1, w2, b2, out):
    ns = jnp.sum(nsum[...], axis=0)
    nc = jnp.sum(ncnt[...].reshape(NW, NUM_GRAPHS, 16), axis=(0, 2))[:, None]
    est = jnp.sum(esum[...].reshape(NW, D_EDGE + 1, NUM_GRAPHS, 8),
                  axis=(0, 3))
    es = est[:D_EDGE].T
    ec = est[D_EDGE][:, None]
    nmean = ns / jnp.maximum(nc, 1.0)
    emean = es / jnp.maximum(ec, 1.0)
    w = w1[...]
    h = (jnp.dot(nmean, w[:D_FEAT], preferred_element_type=jnp.float32)
         + jnp.dot(emean, w[D_FEAT:], preferred_element_type=jnp.float32)
         + b1[...])
    h = jnp.maximum(h, 0.0)
    out[...] = jnp.dot(h, w2[...], preferred_element_type=jnp.float32) + b2[...]


def kernel(x, edge_index, edge_attr, batch, W1, b1, W2, b2):
    ei = edge_index[0].astype(jnp.int32)
    b = batch.astype(jnp.int32)
    ea_t = edge_attr.T
    nsum, ncnt, esum = _sc_pool(x, ei, ea_t, b)
    out = pl.pallas_call(
        _mlp_body,
        out_shape=jax.ShapeDtypeStruct((NUM_GRAPHS, OUT_DIM), jnp.float32),
    )(nsum, ncnt, esum, W1, b1.reshape(1, HIDDEN), W2,
      b2.reshape(1, OUT_DIM))
    return out
